# Initial kernel scaffold; baseline (speedup 1.0000x reference)
import jax
import jax.numpy as jnp
from jax.experimental import pallas as pl


def kernel(x, aa_table, pos_table):
    raise NotImplementedError("write your pallas kernel here")


# speedup vs baseline: 5.1075x; 5.1075x over previous
import functools

import jax
import jax.numpy as jnp
from jax import lax
from jax.experimental import pallas as pl
from jax.experimental.pallas import tpu as pltpu
from jax.experimental.pallas import tpu_sc as plsc

EMB = 128
VOCAB = 27
PEP = 31
BATCH = 16384
ROWS = BATCH * PEP
NC, NS = 2, 16
NW = NC * NS
RPW = ROWS // NW
K = 128
NCHUNK = RPW // K


def _prep_body(x_ref, aa_ref, pos_ref, comb_ref, idx_ref):
    comb_ref[...] = aa_ref[...][:, None, :] + pos_ref[...][None, :, :]
    r = lax.broadcasted_iota(jnp.int32, (ROWS,), 0)
    p = r - (r // PEP) * PEP
    idx_ref[...] = x_ref[...] * PEP + p


def _prep(x1d, aa_table, pos_table):
    return pl.pallas_call(
        _prep_body,
        out_shape=(
            jax.ShapeDtypeStruct((VOCAB, PEP, EMB), jnp.float32),
            jax.ShapeDtypeStruct((ROWS,), jnp.int32),
        ),
    )(x1d, aa_table, pos_table)


def _sc_gather(comb, idx):
    mesh = plsc.VectorSubcoreMesh(core_axis_name="c", subcore_axis_name="s")

    @functools.partial(
        pl.kernel,
        mesh=mesh,
        out_type=jax.ShapeDtypeStruct((ROWS, EMB), jnp.float32),
        scratch_types=[
            pltpu.VMEM((K,), jnp.int32),
            pltpu.VMEM((K, EMB), jnp.float32),
            pltpu.SemaphoreType.DMA,
        ],
    )
    def k(comb_hbm, idx_hbm, out_hbm, idx_v, rows_v, sem):
        wid = lax.axis_index("s") * NC + lax.axis_index("c")
        base = wid * RPW

        def body(j, _):
            off = base + j * K
            pltpu.sync_copy(idx_hbm.at[pl.ds(off, K)], idx_v)
            pltpu.async_copy(comb_hbm.at[idx_v], rows_v, sem).wait()
            pltpu.sync_copy(rows_v, out_hbm.at[pl.ds(off, K)])
            return 0

        lax.fori_loop(0, NCHUNK, body, 0)

    return k(comb, idx)


def kernel(x, aa_table, pos_table):
    x1d = x.reshape(ROWS).astype(jnp.int32)
    comb3, idx = _prep(x1d, aa_table, pos_table)
    comb = comb3.reshape(VOCAB * PEP, EMB)
    out = _sc_gather(comb, idx)
    return out.reshape(BATCH, PEP, EMB)


# speedup vs baseline: 5.4045x; 1.0582x over previous
import functools

import jax
import jax.numpy as jnp
from jax import lax
from jax.experimental import pallas as pl
from jax.experimental.pallas import tpu as pltpu
from jax.experimental.pallas import tpu_sc as plsc

EMB = 128
VOCAB = 27
PEP = 31
BATCH = 16384
ROWS = BATCH * PEP
NC, NS = 2, 16
NW = NC * NS
RPW = ROWS // NW
K = 128
NCHUNK = RPW // K
NBUF = 4


def _prep_body(x_ref, aa_ref, pos_ref, comb_ref, idx_ref):
    comb_ref[...] = aa_ref[...][:, None, :] + pos_ref[...][None, :, :]
    r = lax.broadcasted_iota(jnp.int32, (ROWS,), 0)
    p = r - (r // PEP) * PEP
    idx_ref[...] = x_ref[...] * PEP + p


def _prep(x1d, aa_table, pos_table):
    return pl.pallas_call(
        _prep_body,
        out_shape=(
            jax.ShapeDtypeStruct((VOCAB, PEP, EMB), jnp.float32),
            jax.ShapeDtypeStruct((ROWS,), jnp.int32),
        ),
    )(x1d, aa_table, pos_table)


def _sc_gather(comb, idx2d):
    mesh = plsc.VectorSubcoreMesh(core_axis_name="c", subcore_axis_name="s")

    @functools.partial(
        pl.kernel,
        mesh=mesh,
        out_type=jax.ShapeDtypeStruct((ROWS, EMB), jnp.float32),
        scratch_types=[
            pltpu.VMEM((NCHUNK, K), jnp.int32),
            *[pltpu.VMEM((K, EMB), jnp.float32) for _ in range(NBUF)],
            pltpu.SemaphoreType.DMA((NBUF,)),
            pltpu.SemaphoreType.DMA((NBUF,)),
        ],
    )
    def k(comb_hbm, idx_hbm, out_hbm, idx_all, r0, r1, r2, r3, gsem, osem):
        rows = [r0, r1, r2, r3]
        wid = lax.axis_index("s") * NC + lax.axis_index("c")
        pltpu.sync_copy(idx_hbm.at[wid], idx_all)
        base = wid * RPW

        def wait_gather(s):
            pltpu.make_async_copy(
                comb_hbm.at[idx_all.at[0]], rows[s], gsem.at[s]).wait()

        def wait_out(s):
            pltpu.make_async_copy(
                rows[s], out_hbm.at[pl.ds(base, K)], osem.at[s]).wait()

        def start_gather(j, s):
            pltpu.async_copy(comb_hbm.at[idx_all.at[j]], rows[s], gsem.at[s])

        def start_out(j, s):
            pltpu.async_copy(
                rows[s], out_hbm.at[pl.ds(base + j * K, K)], osem.at[s])

        def body(g, _):
            for s in range(NBUF):
                j = g * NBUF + s
                pl.when(g > 0)(lambda s=s: wait_out(s))
                start_gather(j, s)
                ps = (s - 1) % NBUF
                if s == 0:
                    def prev(g=g, ps=ps):
                        wait_gather(ps)
                        start_out(g * NBUF - 1, ps)
                    pl.when(g > 0)(prev)
                else:
                    wait_gather(ps)
                    start_out(j - 1, ps)
            return 0

        lax.fori_loop(0, NCHUNK // NBUF, body, 0)
        wait_gather(NBUF - 1)
        start_out(NCHUNK - 1, NBUF - 1)
        for s in range(NBUF):
            wait_out(s)

    return k(comb, idx2d)


def kernel(x, aa_table, pos_table):
    x1d = x.reshape(ROWS).astype(jnp.int32)
    comb3, idx = _prep(x1d, aa_table, pos_table)
    comb = comb3.reshape(VOCAB * PEP, EMB)
    out = _sc_gather(comb, idx.reshape(NW, NCHUNK, K))
    return out.reshape(BATCH, PEP, EMB)
